# baseline (speedup 1.0000x reference)
import jax
import jax.numpy as jnp
from jax import lax
from jax.experimental import pallas as pl
from jax.experimental.pallas import tpu as pltpu
from jax.experimental.pallas import tpu_sc as plsc

DIM_ = 128
B_ = 16384
NC_ = 2
NS_ = 16
NW_ = NC_ * NS_
NCHUNK_ = 4
CB_ = B_ // NCHUNK_
BPW_ = CB_ // NW_


def _sc_gather_body(idx_hbm, t0_hbm, t1_hbm, t2_hbm, t3_hbm, out_hbm,
                    idx_v, rows_v, gsem, wsem):
    wid = lax.axis_index("s") * NC_ + lax.axis_index("c")
    base = wid * BPW_
    pltpu.sync_copy(idx_hbm.at[wid], idx_v)
    tabs = (t0_hbm, t1_hbm, t2_hbm, t3_hbm)
    gathers = [
        pltpu.async_copy(tabs[t].at[idx_v.at[t]], rows_v.at[t], gsem)
        for t in range(4)
    ]
    writes = []
    for t in range(4):
        gathers[t].wait()
        writes.append(
            pltpu.async_copy(rows_v.at[t], out_hbm.at[t, pl.ds(base, BPW_)],
                             wsem))
    for w in writes:
        w.wait()


_gather4 = pl.kernel(
    _sc_gather_body,
    out_type=jax.ShapeDtypeStruct((4, CB_, DIM_), jnp.float32),
    mesh=plsc.VectorSubcoreMesh(core_axis_name="c", subcore_axis_name="s"),
    scratch_types=[
        pltpu.VMEM((4, BPW_), jnp.int32),
        pltpu.VMEM((4, BPW_, DIM_), jnp.float32),
        pltpu.SemaphoreType.DMA,
        pltpu.SemaphoreType.DMA,
    ],
)


def _proj_body(x_ref, w_ref, b_ref, o_ref):
    acc = jnp.broadcast_to(b_ref[...], o_ref.shape).astype(jnp.float32)
    for t in range(4):
        acc = acc + jnp.dot(x_ref[t], w_ref[t],
                            preferred_element_type=jnp.float32)
    o_ref[...] = acc


def _proj(x, w4, b2, bm=1024):
    return pl.pallas_call(
        _proj_body,
        grid=(CB_ // bm,),
        in_specs=[
            pl.BlockSpec((4, bm, DIM_), lambda i: (0, i, 0)),
            pl.BlockSpec((4, DIM_, DIM_), lambda i: (0, 0, 0)),
            pl.BlockSpec((1, DIM_), lambda i: (0, 0)),
        ],
        out_specs=pl.BlockSpec((bm, DIM_), lambda i: (i, 0)),
        out_shape=jax.ShapeDtypeStruct((CB_, DIM_), jnp.float32),
    )(x, w4, b2)


def kernel(cell_type, dose, time, batch, E_cell_type, E_dose, E_time,
           E_batch, W, b):
    idx = jnp.stack([cell_type.astype(jnp.int32), dose.astype(jnp.int32),
                     time.astype(jnp.int32), batch.astype(jnp.int32)])
    idx = idx.reshape(4, NCHUNK_, NW_, BPW_).transpose(1, 2, 0, 3)
    w4 = W.reshape(4, DIM_, DIM_)
    b2 = b.reshape(1, DIM_)
    outs = []
    for c in range(NCHUNK_):
        gathered = _gather4(idx[c], E_cell_type, E_dose, E_time, E_batch)
        outs.append(_proj(gathered, w4, b2))
    return jnp.concatenate(outs, axis=0)


# speedup vs baseline: 1.1070x; 1.1070x over previous
import jax
import jax.numpy as jnp
from jax import lax
from jax.experimental import pallas as pl
from jax.experimental.pallas import tpu as pltpu
from jax.experimental.pallas import tpu_sc as plsc

DIM_ = 128
B_ = 16384
NC_ = 2
NS_ = 16
NW_ = NC_ * NS_
NCHUNK_ = 4
CB_ = B_ // NCHUNK_
BPW_ = CB_ // NW_


def _preproj_body(e_ref, w_ref, b_ref, o_ref):
    o_ref[...] = (jnp.dot(e_ref[...], w_ref[...],
                          preferred_element_type=jnp.float32)
                  + b_ref[...])


def _preproj(e, w, b2):
    return pl.pallas_call(
        _preproj_body,
        out_shape=jax.ShapeDtypeStruct(e.shape, jnp.float32),
    )(e, w, b2)


def _sc_gather_body(idx_hbm, tc_hbm, tb_hbm, p1_hbm, p2_hbm, out_hbm,
                    idx_v, rows_v, s1_v, s2_v, gsem, wsem):
    wid = lax.axis_index("s") * NC_ + lax.axis_index("c")
    base = wid * BPW_
    pltpu.sync_copy(idx_hbm.at[wid], idx_v)
    g1 = pltpu.async_copy(p1_hbm.at[idx_v.at[1]], s1_v, gsem)
    g2 = pltpu.async_copy(p2_hbm.at[idx_v.at[2]], s2_v, gsem)
    g0 = pltpu.async_copy(tc_hbm.at[idx_v.at[0]], rows_v.at[0], gsem)
    g3 = pltpu.async_copy(tb_hbm.at[idx_v.at[3]], rows_v.at[1], gsem)
    g1.wait()
    g2.wait()

    def _add_row(r, _):
        for c in range(DIM_ // 16):
            plsc.addupdate(s1_v.at[r, pl.ds(c * 16, 16)],
                           s2_v[r, pl.ds(c * 16, 16)])
        return _

    lax.fori_loop(0, BPW_, _add_row, 0, unroll=2)
    ws = pltpu.async_copy(s1_v, out_hbm.at[2, pl.ds(base, BPW_)], wsem)
    g0.wait()
    w0 = pltpu.async_copy(rows_v.at[0], out_hbm.at[0, pl.ds(base, BPW_)],
                          wsem)
    g3.wait()
    w1 = pltpu.async_copy(rows_v.at[1], out_hbm.at[1, pl.ds(base, BPW_)],
                          wsem)
    ws.wait()
    w0.wait()
    w1.wait()


_gather4 = pl.kernel(
    _sc_gather_body,
    out_type=jax.ShapeDtypeStruct((3, CB_, DIM_), jnp.float32),
    mesh=plsc.VectorSubcoreMesh(core_axis_name="c", subcore_axis_name="s"),
    scratch_types=[
        pltpu.VMEM((4, BPW_), jnp.int32),
        pltpu.VMEM((2, BPW_, DIM_), jnp.float32),
        pltpu.VMEM((BPW_, DIM_), jnp.float32),
        pltpu.VMEM((BPW_, DIM_), jnp.float32),
        pltpu.SemaphoreType.DMA,
        pltpu.SemaphoreType.DMA,
    ],
)


def _proj_body(c, bm, x_ref, w_ref, buf_ref, o_hbm, acc_v, sem):
    i = pl.program_id(0)
    acc_v[...] = (x_ref[2]
                  + jnp.dot(x_ref[0], w_ref[0],
                            preferred_element_type=jnp.float32)
                  + jnp.dot(x_ref[1], w_ref[1],
                            preferred_element_type=jnp.float32))
    cp = pltpu.make_async_copy(
        acc_v, o_hbm.at[pl.ds(c * CB_ + i * bm, bm), :], sem)
    cp.start()
    cp.wait()


def _proj(c, x, w2, buf, bm=1024):
    import functools
    return pl.pallas_call(
        functools.partial(_proj_body, c, bm),
        grid=(CB_ // bm,),
        in_specs=[
            pl.BlockSpec((3, bm, DIM_), lambda i: (0, i, 0)),
            pl.BlockSpec((2, DIM_, DIM_), lambda i: (0, 0, 0)),
            pl.BlockSpec(memory_space=pl.ANY),
        ],
        out_specs=pl.BlockSpec(memory_space=pl.ANY),
        out_shape=jax.ShapeDtypeStruct((B_, DIM_), jnp.float32),
        scratch_shapes=[
            pltpu.VMEM((bm, DIM_), jnp.float32),
            pltpu.SemaphoreType.DMA,
        ],
        input_output_aliases={2: 0},
    )(x, w2, buf)


def kernel(cell_type, dose, time, batch, E_cell_type, E_dose, E_time,
           E_batch, W, b):
    idx = jnp.stack([cell_type.astype(jnp.int32), dose.astype(jnp.int32),
                     time.astype(jnp.int32), batch.astype(jnp.int32)])
    idx = idx.reshape(4, NCHUNK_, NW_, BPW_).transpose(1, 2, 0, 3)
    w4 = W.reshape(4, DIM_, DIM_)
    b2 = b.reshape(1, DIM_)
    p1 = _preproj(E_dose, w4[1], b2)
    p2 = _preproj(E_time, w4[2], jnp.zeros_like(b2))
    w2 = jnp.stack([w4[0], w4[3]])
    buf = jnp.zeros((B_, DIM_), jnp.float32)
    for c in range(NCHUNK_):
        gathered = _gather4(idx[c], E_cell_type, E_batch, p1, p2)
        buf = _proj(c, gathered, w2, buf)
    return buf


# speedup vs baseline: 1.1826x; 1.0684x over previous
import functools

import jax
import jax.numpy as jnp
from jax import lax
from jax.experimental import pallas as pl
from jax.experimental.pallas import tpu as pltpu
from jax.experimental.pallas import tpu_sc as plsc

DIM_ = 128
B_ = 16384
NC_ = 2
NS_ = 16
NW_ = NC_ * NS_
NCHUNK_ = 4
CB_ = B_ // NCHUNK_
BPW_ = CB_ // NW_
BM_ = 512
NB_ = CB_ // BM_


def _preproj_body(ed_ref, et_ref, w_ref, b_ref, o1_ref, o2_ref):
    o1_ref[...] = (jnp.dot(ed_ref[...], w_ref[pl.ds(DIM_, DIM_), :],
                           preferred_element_type=jnp.float32)
                   + b_ref[...])
    o2_ref[...] = jnp.dot(et_ref[...], w_ref[pl.ds(2 * DIM_, DIM_), :],
                          preferred_element_type=jnp.float32)


def _preproj(e_dose, e_time, w, b2):
    n = e_dose.shape[0]
    sds = jax.ShapeDtypeStruct((n, DIM_), jnp.float32)
    return pl.pallas_call(
        _preproj_body,
        out_shape=[sds, sds],
    )(e_dose, e_time, w, b2)


def _sc_gather_body(c, ic_hbm, id_hbm, it_hbm, ib_hbm, tc_hbm, tb_hbm,
                    p1_hbm, p2_hbm, out_hbm, idx_v, rows_v, s1_v, s2_v,
                    isem, gsem, wsem):
    wid = lax.axis_index("s") * NC_ + lax.axis_index("c")
    base = wid * BPW_
    src = c * CB_ + base
    ics = [
        pltpu.async_copy(h.at[pl.ds(src, BPW_)], idx_v.at[t], isem)
        for t, h in enumerate((id_hbm, it_hbm, ic_hbm, ib_hbm))
    ]
    for ic in ics:
        ic.wait()
    g1 = pltpu.async_copy(p1_hbm.at[idx_v.at[0]], s1_v, gsem)
    g2 = pltpu.async_copy(p2_hbm.at[idx_v.at[1]], s2_v, gsem)
    g0 = pltpu.async_copy(tc_hbm.at[idx_v.at[2]], rows_v.at[0], gsem)
    g3 = pltpu.async_copy(tb_hbm.at[idx_v.at[3]], rows_v.at[1], gsem)
    g1.wait()
    g2.wait()

    def _add_row(r, carry):
        for k in range(DIM_ // 16):
            plsc.addupdate(s1_v.at[r, pl.ds(k * 16, 16)],
                           s2_v[r, pl.ds(k * 16, 16)])
        return carry

    lax.fori_loop(0, BPW_, _add_row, 0, unroll=2)
    ws = pltpu.async_copy(s1_v, out_hbm.at[2, pl.ds(base, BPW_)], wsem)
    g0.wait()
    w0 = pltpu.async_copy(rows_v.at[0], out_hbm.at[0, pl.ds(base, BPW_)],
                          wsem)
    g3.wait()
    w1 = pltpu.async_copy(rows_v.at[1], out_hbm.at[1, pl.ds(base, BPW_)],
                          wsem)
    ws.wait()
    w0.wait()
    w1.wait()


def _make_gather(c):
    return pl.kernel(
        functools.partial(_sc_gather_body, c),
        out_type=jax.ShapeDtypeStruct((3, CB_, DIM_), jnp.float32),
        mesh=plsc.VectorSubcoreMesh(core_axis_name="c",
                                    subcore_axis_name="s"),
        scratch_types=[
            pltpu.VMEM((4, BPW_), jnp.int32),
            pltpu.VMEM((2, BPW_, DIM_), jnp.float32),
            pltpu.VMEM((BPW_, DIM_), jnp.float32),
            pltpu.VMEM((BPW_, DIM_), jnp.float32),
            pltpu.SemaphoreType.DMA,
            pltpu.SemaphoreType.DMA,
            pltpu.SemaphoreType.DMA,
        ],
    )


_gathers = [_make_gather(c) for c in range(NCHUNK_)]


def _proj_compute(x_ref, w_ref):
    return (x_ref[2]
            + jnp.dot(x_ref[0], w_ref[pl.ds(0, DIM_), :],
                      preferred_element_type=jnp.float32)
            + jnp.dot(x_ref[1], w_ref[pl.ds(3 * DIM_, DIM_), :],
                      preferred_element_type=jnp.float32))


def _proj_body(c, x_ref, w_ref, o_hbm, acc_v, sem):
    i = pl.program_id(0)
    slot = lax.rem(i, 2)
    acc_v[slot] = _proj_compute(x_ref, w_ref)

    @pl.when(i > 0)
    def _wait_prev():
        pltpu.make_async_copy(
            acc_v.at[1 - slot],
            o_hbm.at[pl.ds(c * CB_ + (i - 1) * BM_, BM_), :], sem).wait()

    cp = pltpu.make_async_copy(
        acc_v.at[slot], o_hbm.at[pl.ds(c * CB_ + i * BM_, BM_), :], sem)
    cp.start()

    @pl.when(i == NB_ - 1)
    def _wait_last():
        cp.wait()


def _proj_body_alias(c, x_ref, w_ref, buf_ref, o_hbm, acc_v, sem):
    _proj_body(c, x_ref, w_ref, o_hbm, acc_v, sem)


def _proj(c, x, w, buf):
    common = dict(
        grid=(NB_,),
        out_specs=pl.BlockSpec(memory_space=pl.ANY),
        out_shape=jax.ShapeDtypeStruct((B_, DIM_), jnp.float32),
        scratch_shapes=[
            pltpu.VMEM((2, BM_, DIM_), jnp.float32),
            pltpu.SemaphoreType.DMA,
        ],
    )
    x_spec = pl.BlockSpec((3, BM_, DIM_), lambda i: (0, i, 0))
    w_spec = pl.BlockSpec((4 * DIM_, DIM_), lambda i: (0, 0))
    if buf is None:
        return pl.pallas_call(
            functools.partial(_proj_body, c),
            in_specs=[x_spec, w_spec],
            **common,
        )(x, w)
    return pl.pallas_call(
        functools.partial(_proj_body_alias, c),
        in_specs=[x_spec, w_spec, pl.BlockSpec(memory_space=pl.ANY)],
        input_output_aliases={2: 0},
        **common,
    )(x, w, buf)


def kernel(cell_type, dose, time, batch, E_cell_type, E_dose, E_time,
           E_batch, W, b):
    ic = cell_type.astype(jnp.int32)
    id_ = dose.astype(jnp.int32)
    it = time.astype(jnp.int32)
    ib = batch.astype(jnp.int32)
    p1, p2 = _preproj(E_dose, E_time, W, b.reshape(1, DIM_))
    buf = None
    for c in range(NCHUNK_):
        gathered = _gathers[c](ic, id_, it, ib, E_cell_type, E_batch,
                               p1, p2)
        buf = _proj(c, gathered, W, buf)
    return buf


# speedup vs baseline: 1.1853x; 1.0023x over previous
import functools

import jax
import jax.numpy as jnp
from jax import lax
from jax.experimental import pallas as pl
from jax.experimental.pallas import tpu as pltpu
from jax.experimental.pallas import tpu_sc as plsc

DIM_ = 128
B_ = 16384
NC_ = 2
NS_ = 16
NW_ = NC_ * NS_
NCHUNK_ = 4
CB_ = B_ // NCHUNK_
BPW_ = CB_ // NW_
BM_ = 512
NB_ = CB_ // BM_


def _preproj_body(ed_ref, et_ref, w_ref, b_ref, o1_ref, o2_ref):
    o1_ref[...] = (jnp.dot(ed_ref[...], w_ref[pl.ds(DIM_, DIM_), :],
                           preferred_element_type=jnp.float32)
                   + b_ref[...])
    o2_ref[...] = jnp.dot(et_ref[...], w_ref[pl.ds(2 * DIM_, DIM_), :],
                          preferred_element_type=jnp.float32)


def _preproj(e_dose, e_time, w, b2):
    n = e_dose.shape[0]
    sds = jax.ShapeDtypeStruct((n, DIM_), jnp.float32)
    return pl.pallas_call(
        _preproj_body,
        out_shape=[sds, sds],
    )(e_dose, e_time, w, b2)


def _sc_gather_body(c, ic_hbm, id_hbm, it_hbm, ib_hbm, tc_hbm, tb_hbm,
                    p1_hbm, p2_hbm, out_hbm, idx_v, rows_v, s1_v, s2_v,
                    isem, gsem, wsem):
    wid = lax.axis_index("s") * NC_ + lax.axis_index("c")
    base = wid * BPW_
    src = c * CB_ + base
    ics = [
        pltpu.async_copy(h.at[pl.ds(src, BPW_)], idx_v.at[t], isem)
        for t, h in enumerate((id_hbm, it_hbm, ic_hbm, ib_hbm))
    ]
    for ic in ics:
        ic.wait()
    g1 = pltpu.async_copy(p1_hbm.at[idx_v.at[0]], s1_v, gsem)
    g2 = pltpu.async_copy(p2_hbm.at[idx_v.at[1]], s2_v, gsem)
    g0 = pltpu.async_copy(tc_hbm.at[idx_v.at[2]], rows_v.at[0], gsem)
    g3 = pltpu.async_copy(tb_hbm.at[idx_v.at[3]], rows_v.at[1], gsem)
    g1.wait()
    g2.wait()

    def _add_row(r, carry):
        for k in range(DIM_ // 16):
            plsc.addupdate(s1_v.at[r, pl.ds(k * 16, 16)],
                           s2_v[r, pl.ds(k * 16, 16)])
        return carry

    lax.fori_loop(0, BPW_, _add_row, 0, unroll=2)
    ws = pltpu.async_copy(s1_v, out_hbm.at[2, pl.ds(base, BPW_)], wsem)
    g0.wait()
    w0 = pltpu.async_copy(rows_v.at[0], out_hbm.at[0, pl.ds(base, BPW_)],
                          wsem)
    g3.wait()
    w1 = pltpu.async_copy(rows_v.at[1], out_hbm.at[1, pl.ds(base, BPW_)],
                          wsem)
    ws.wait()
    w0.wait()
    w1.wait()


def _make_gather(c):
    return pl.kernel(
        functools.partial(_sc_gather_body, c),
        out_type=jax.ShapeDtypeStruct((3, CB_, DIM_), jnp.float32),
        mesh=plsc.VectorSubcoreMesh(core_axis_name="c",
                                    subcore_axis_name="s"),
        scratch_types=[
            pltpu.VMEM((4, BPW_), jnp.int32),
            pltpu.VMEM((2, BPW_, DIM_), jnp.float32),
            pltpu.VMEM((BPW_, DIM_), jnp.float32),
            pltpu.VMEM((BPW_, DIM_), jnp.float32),
            pltpu.SemaphoreType.DMA,
            pltpu.SemaphoreType.DMA,
            pltpu.SemaphoreType.DMA,
        ],
    )


_gathers = [_make_gather(c) for c in range(NCHUNK_)]


def _proj_compute(x_ref, w_ref):
    return (x_ref[2]
            + jnp.dot(x_ref[0], w_ref[pl.ds(0, DIM_), :],
                      preferred_element_type=jnp.float32)
            + jnp.dot(x_ref[1], w_ref[pl.ds(3 * DIM_, DIM_), :],
                      preferred_element_type=jnp.float32))


def _proj_body(x_ref, w_ref, o_ref):
    o_ref[...] = _proj_compute(x_ref, w_ref)


def _proj_body_alias(x_ref, w_ref, buf_ref, o_ref):
    o_ref[...] = _proj_compute(x_ref, w_ref)


def _proj(c, x, w, buf):
    common = dict(
        grid=(NB_,),
        out_specs=pl.BlockSpec((BM_, DIM_), lambda i: (c * NB_ + i, 0)),
        out_shape=jax.ShapeDtypeStruct((B_, DIM_), jnp.float32),
    )
    x_spec = pl.BlockSpec((3, BM_, DIM_), lambda i: (0, i, 0))
    w_spec = pl.BlockSpec((4 * DIM_, DIM_), lambda i: (0, 0))
    if buf is None:
        return pl.pallas_call(
            _proj_body,
            in_specs=[x_spec, w_spec],
            **common,
        )(x, w)
    return pl.pallas_call(
        _proj_body_alias,
        in_specs=[x_spec, w_spec, pl.BlockSpec(memory_space=pl.ANY)],
        input_output_aliases={2: 0},
        **common,
    )(x, w, buf)


def kernel(cell_type, dose, time, batch, E_cell_type, E_dose, E_time,
           E_batch, W, b):
    ic = cell_type.astype(jnp.int32)
    id_ = dose.astype(jnp.int32)
    it = time.astype(jnp.int32)
    ib = batch.astype(jnp.int32)
    p1, p2 = _preproj(E_dose, E_time, W, b.reshape(1, DIM_))
    buf = None
    for c in range(NCHUNK_):
        gathered = _gathers[c](ic, id_, it, ib, E_cell_type, E_batch,
                               p1, p2)
        buf = _proj(c, gathered, W, buf)
    return buf


# speedup vs baseline: 1.2276x; 1.0357x over previous
import functools

import jax
import jax.numpy as jnp
from jax import lax
from jax.experimental import pallas as pl
from jax.experimental.pallas import tpu as pltpu
from jax.experimental.pallas import tpu_sc as plsc

DIM_ = 128
B_ = 16384
NC_ = 2
NS_ = 16
NW_ = NC_ * NS_
NCHUNK_ = 4
CB_ = B_ // NCHUNK_
BPW_ = CB_ // NW_
BM_ = 1024
NB_ = CB_ // BM_


def _preproj_body(ed_ref, et_ref, w_ref, b_ref, o1_ref, o2_ref):
    o1_ref[...] = (jnp.dot(ed_ref[...], w_ref[pl.ds(DIM_, DIM_), :],
                           preferred_element_type=jnp.float32)
                   + b_ref[...])
    o2_ref[...] = jnp.dot(et_ref[...], w_ref[pl.ds(2 * DIM_, DIM_), :],
                          preferred_element_type=jnp.float32)


def _preproj(e_dose, e_time, w, b2):
    n = e_dose.shape[0]
    sds = jax.ShapeDtypeStruct((n, DIM_), jnp.float32)
    return pl.pallas_call(
        _preproj_body,
        out_shape=[sds, sds],
    )(e_dose, e_time, w, b2)


def _sc_gather_body(c, ic_hbm, id_hbm, it_hbm, ib_hbm, tc_hbm, tb_hbm,
                    p1_hbm, p2_hbm, out_hbm, idx_v, rows_v, s1_v, s2_v,
                    isem, gsem, wsem):
    wid = lax.axis_index("s") * NC_ + lax.axis_index("c")
    base = wid * BPW_
    src = c * CB_ + base
    ics = [
        pltpu.async_copy(h.at[pl.ds(src, BPW_)], idx_v.at[t], isem)
        for t, h in enumerate((id_hbm, it_hbm, ic_hbm, ib_hbm))
    ]
    for ic in ics:
        ic.wait()
    g1 = pltpu.async_copy(p1_hbm.at[idx_v.at[0]], s1_v, gsem)
    g2 = pltpu.async_copy(p2_hbm.at[idx_v.at[1]], s2_v, gsem)
    g0 = pltpu.async_copy(tc_hbm.at[idx_v.at[2]], rows_v.at[0], gsem)
    g3 = pltpu.async_copy(tb_hbm.at[idx_v.at[3]], rows_v.at[1], gsem)
    g1.wait()
    g2.wait()

    def _add_row(r, carry):
        for k in range(DIM_ // 16):
            plsc.addupdate(s1_v.at[r, pl.ds(k * 16, 16)],
                           s2_v[r, pl.ds(k * 16, 16)])
        return carry

    lax.fori_loop(0, BPW_, _add_row, 0, unroll=2)
    ws = pltpu.async_copy(s1_v, out_hbm.at[2, pl.ds(base, BPW_)], wsem)
    g0.wait()
    w0 = pltpu.async_copy(rows_v.at[0], out_hbm.at[0, pl.ds(base, BPW_)],
                          wsem)
    g3.wait()
    w1 = pltpu.async_copy(rows_v.at[1], out_hbm.at[1, pl.ds(base, BPW_)],
                          wsem)
    ws.wait()
    w0.wait()
    w1.wait()


def _make_gather(c):
    return pl.kernel(
        functools.partial(_sc_gather_body, c),
        out_type=jax.ShapeDtypeStruct((3, CB_, DIM_), jnp.float32),
        mesh=plsc.VectorSubcoreMesh(core_axis_name="c",
                                    subcore_axis_name="s"),
        scratch_types=[
            pltpu.VMEM((4, BPW_), jnp.int32),
            pltpu.VMEM((2, BPW_, DIM_), jnp.float32),
            pltpu.VMEM((BPW_, DIM_), jnp.float32),
            pltpu.VMEM((BPW_, DIM_), jnp.float32),
            pltpu.SemaphoreType.DMA,
            pltpu.SemaphoreType.DMA,
            pltpu.SemaphoreType.DMA,
        ],
    )


_gathers = [_make_gather(c) for c in range(NCHUNK_)]


def _proj_compute(x_ref, w_ref):
    return (x_ref[2]
            + jnp.dot(x_ref[0], w_ref[pl.ds(0, DIM_), :],
                      preferred_element_type=jnp.float32)
            + jnp.dot(x_ref[1], w_ref[pl.ds(3 * DIM_, DIM_), :],
                      preferred_element_type=jnp.float32))


def _proj_body(x_ref, w_ref, o_ref):
    o_ref[...] = _proj_compute(x_ref, w_ref)


def _proj_body_alias(x_ref, w_ref, buf_ref, o_ref):
    o_ref[...] = _proj_compute(x_ref, w_ref)


def _proj(c, x, w, buf):
    common = dict(
        grid=(NB_,),
        out_specs=pl.BlockSpec((BM_, DIM_), lambda i: (c * NB_ + i, 0)),
        out_shape=jax.ShapeDtypeStruct((B_, DIM_), jnp.float32),
    )
    x_spec = pl.BlockSpec((3, BM_, DIM_), lambda i: (0, i, 0))
    w_spec = pl.BlockSpec((4 * DIM_, DIM_), lambda i: (0, 0))
    if buf is None:
        return pl.pallas_call(
            _proj_body,
            in_specs=[x_spec, w_spec],
            **common,
        )(x, w)
    return pl.pallas_call(
        _proj_body_alias,
        in_specs=[x_spec, w_spec, pl.BlockSpec(memory_space=pl.ANY)],
        input_output_aliases={2: 0},
        **common,
    )(x, w, buf)


def kernel(cell_type, dose, time, batch, E_cell_type, E_dose, E_time,
           E_batch, W, b):
    ic = cell_type.astype(jnp.int32)
    id_ = dose.astype(jnp.int32)
    it = time.astype(jnp.int32)
    ib = batch.astype(jnp.int32)
    p1, p2 = _preproj(E_dose, E_time, W, b.reshape(1, DIM_))
    buf = None
    for c in range(NCHUNK_):
        gathered = _gathers[c](ic, id_, it, ib, E_cell_type, E_batch,
                               p1, p2)
        buf = _proj(c, gathered, W, buf)
    return buf


# speedup vs baseline: 1.2287x; 1.0009x over previous
import functools

import jax
import jax.numpy as jnp
from jax import lax
from jax.experimental import pallas as pl
from jax.experimental.pallas import tpu as pltpu
from jax.experimental.pallas import tpu_sc as plsc

DIM_ = 128
B_ = 16384
NC_ = 2
NS_ = 16
NW_ = NC_ * NS_
NCHUNK_ = 4
CB_ = B_ // NCHUNK_
BPW_ = CB_ // NW_
BM_ = 1024
NB_ = CB_ // BM_


def _preproj_body(ed_ref, et_ref, w_ref, b_ref, o1_ref, o2_ref):
    o1_ref[...] = (jnp.dot(ed_ref[...], w_ref[pl.ds(DIM_, DIM_), :],
                           preferred_element_type=jnp.float32)
                   + b_ref[...])
    o2_ref[...] = jnp.dot(et_ref[...], w_ref[pl.ds(2 * DIM_, DIM_), :],
                          preferred_element_type=jnp.float32)


def _preproj(e_dose, e_time, w, b2):
    n = e_dose.shape[0]
    sds = jax.ShapeDtypeStruct((n, DIM_), jnp.float32)
    return pl.pallas_call(
        _preproj_body,
        out_shape=[sds, sds],
    )(e_dose, e_time, w, b2)


def _sc_gather_body(c, ic_hbm, id_hbm, it_hbm, ib_hbm, tc_hbm, tb_hbm,
                    p1_hbm, p2_hbm, out_hbm, idx_v, rows_v, s1_v, s2_v,
                    isem, gsem, wsem):
    wid = lax.axis_index("s") * NC_ + lax.axis_index("c")
    base = wid * BPW_
    src = c * CB_ + base
    ics = [
        pltpu.async_copy(h.at[pl.ds(src, BPW_)], idx_v.at[t], isem)
        for t, h in enumerate((id_hbm, it_hbm, ic_hbm, ib_hbm))
    ]
    for ic in ics:
        ic.wait()
    g1 = pltpu.async_copy(p1_hbm.at[idx_v.at[0]], s1_v, gsem)
    g2 = pltpu.async_copy(p2_hbm.at[idx_v.at[1]], s2_v, gsem)
    g0 = pltpu.async_copy(tc_hbm.at[idx_v.at[2]], rows_v.at[0], gsem)
    g3 = pltpu.async_copy(tb_hbm.at[idx_v.at[3]], rows_v.at[1], gsem)
    g1.wait()
    g2.wait()

    def _add_row(r, carry):
        for k in range(DIM_ // 16):
            plsc.addupdate(s1_v.at[r, pl.ds(k * 16, 16)],
                           s2_v[r, pl.ds(k * 16, 16)])
        return carry

    lax.fori_loop(0, BPW_, _add_row, 0, unroll=2)
    ws = pltpu.async_copy(s1_v, out_hbm.at[2, pl.ds(base, BPW_)], wsem)
    g0.wait()
    w0 = pltpu.async_copy(rows_v.at[0], out_hbm.at[0, pl.ds(base, BPW_)],
                          wsem)
    g3.wait()
    w1 = pltpu.async_copy(rows_v.at[1], out_hbm.at[1, pl.ds(base, BPW_)],
                          wsem)
    ws.wait()
    w0.wait()
    w1.wait()


def _make_gather(c):
    return pl.kernel(
        functools.partial(_sc_gather_body, c),
        out_type=jax.ShapeDtypeStruct((3, CB_, DIM_), jnp.float32),
        mesh=plsc.VectorSubcoreMesh(core_axis_name="c",
                                    subcore_axis_name="s"),
        scratch_types=[
            pltpu.VMEM((4, BPW_), jnp.int32),
            pltpu.VMEM((2, BPW_, DIM_), jnp.float32),
            pltpu.VMEM((BPW_, DIM_), jnp.float32),
            pltpu.VMEM((BPW_, DIM_), jnp.float32),
            pltpu.SemaphoreType.DMA,
            pltpu.SemaphoreType.DMA,
            pltpu.SemaphoreType.DMA,
        ],
    )


_gathers = [_make_gather(c) for c in range(NCHUNK_)]


def _sc_gather_raw_body(ic_hbm, id_hbm, it_hbm, ib_hbm, t0_hbm, t1_hbm,
                        t2_hbm, t3_hbm, out_hbm, idx_v, rows_v, isem,
                        gsem, wsem):
    wid = lax.axis_index("s") * NC_ + lax.axis_index("c")
    base = wid * BPW_
    ics = [
        pltpu.async_copy(h.at[pl.ds(base, BPW_)], idx_v.at[t], isem)
        for t, h in enumerate((ic_hbm, id_hbm, it_hbm, ib_hbm))
    ]
    for ic in ics:
        ic.wait()
    tabs = (t0_hbm, t1_hbm, t2_hbm, t3_hbm)
    gathers = [
        pltpu.async_copy(tabs[t].at[idx_v.at[t]], rows_v.at[t], gsem)
        for t in range(4)
    ]
    writes = []
    for t in range(4):
        gathers[t].wait()
        writes.append(
            pltpu.async_copy(rows_v.at[t], out_hbm.at[t, pl.ds(base, BPW_)],
                             wsem))
    for w in writes:
        w.wait()


_gather_raw = pl.kernel(
    _sc_gather_raw_body,
    out_type=jax.ShapeDtypeStruct((4, CB_, DIM_), jnp.float32),
    mesh=plsc.VectorSubcoreMesh(core_axis_name="c", subcore_axis_name="s"),
    scratch_types=[
        pltpu.VMEM((4, BPW_), jnp.int32),
        pltpu.VMEM((4, BPW_, DIM_), jnp.float32),
        pltpu.SemaphoreType.DMA,
        pltpu.SemaphoreType.DMA,
        pltpu.SemaphoreType.DMA,
    ],
)


def _proj_compute(x_ref, w_ref):
    return (x_ref[2]
            + jnp.dot(x_ref[0], w_ref[pl.ds(0, DIM_), :],
                      preferred_element_type=jnp.float32)
            + jnp.dot(x_ref[1], w_ref[pl.ds(3 * DIM_, DIM_), :],
                      preferred_element_type=jnp.float32))


def _proj_raw_body(x_ref, w_ref, b_ref, o_ref):
    acc = jnp.broadcast_to(b_ref[...], o_ref.shape).astype(jnp.float32)
    for t in range(4):
        acc = acc + jnp.dot(x_ref[t], w_ref[pl.ds(t * DIM_, DIM_), :],
                            preferred_element_type=jnp.float32)
    o_ref[...] = acc


def _proj_body_alias(x_ref, w_ref, buf_ref, o_ref):
    o_ref[...] = _proj_compute(x_ref, w_ref)


def _proj_raw(x, w, b2):
    return pl.pallas_call(
        _proj_raw_body,
        grid=(NB_,),
        in_specs=[
            pl.BlockSpec((4, BM_, DIM_), lambda i: (0, i, 0)),
            pl.BlockSpec((4 * DIM_, DIM_), lambda i: (0, 0)),
            pl.BlockSpec((1, DIM_), lambda i: (0, 0)),
        ],
        out_specs=pl.BlockSpec((BM_, DIM_), lambda i: (i, 0)),
        out_shape=jax.ShapeDtypeStruct((B_, DIM_), jnp.float32),
    )(x, w, b2)


def _proj(c, x, w, buf):
    return pl.pallas_call(
        _proj_body_alias,
        grid=(NB_,),
        in_specs=[
            pl.BlockSpec((3, BM_, DIM_), lambda i: (0, i, 0)),
            pl.BlockSpec((4 * DIM_, DIM_), lambda i: (0, 0)),
            pl.BlockSpec(memory_space=pl.ANY),
        ],
        out_specs=pl.BlockSpec((BM_, DIM_), lambda i: (c * NB_ + i, 0)),
        out_shape=jax.ShapeDtypeStruct((B_, DIM_), jnp.float32),
        input_output_aliases={2: 0},
    )(x, w, buf)


def kernel(cell_type, dose, time, batch, E_cell_type, E_dose, E_time,
           E_batch, W, b):
    ic = cell_type.astype(jnp.int32)
    id_ = dose.astype(jnp.int32)
    it = time.astype(jnp.int32)
    ib = batch.astype(jnp.int32)
    b2 = b.reshape(1, DIM_)
    g0 = _gather_raw(ic, id_, it, ib, E_cell_type, E_dose, E_time, E_batch)
    p1, p2 = _preproj(E_dose, E_time, W, b2)
    buf = _proj_raw(g0, W, b2)
    for c in range(1, NCHUNK_):
        gathered = _gathers[c](ic, id_, it, ib, E_cell_type, E_batch,
                               p1, p2)
        buf = _proj(c, gathered, W, buf)
    return buf
